# baseline (speedup 1.0000x reference)
import functools

import jax
import jax.numpy as jnp
from jax import lax
from jax.experimental import pallas as pl
from jax.experimental.pallas import tpu as pltpu
from jax.experimental.pallas import tpu_sc as plsc

_NC = 2
_NS = 16
_NW = _NC * _NS
_C = 128


def _transform_table(table, W, b):
    n, d = table.shape
    blk = 1024

    def body(x_ref, w_ref, b_ref, o_ref):
        acc = jnp.dot(x_ref[...], w_ref[...], preferred_element_type=jnp.float32)
        o_ref[...] = jnp.maximum(acc + b_ref[...], 0.0)

    return pl.pallas_call(
        body,
        grid=(pl.cdiv(n, blk),),
        in_specs=[
            pl.BlockSpec((blk, d), lambda i: (i, 0)),
            pl.BlockSpec((d, d), lambda i: (0, 0)),
            pl.BlockSpec((1, d), lambda i: (0, 0)),
        ],
        out_specs=pl.BlockSpec((blk, d), lambda i: (i, 0)),
        out_shape=jax.ShapeDtypeStruct((n, d), jnp.float32),
    )(table, W, b.reshape(1, d))


def _gather_rows(table2, idx3d, total, d):
    chunks = idx3d.shape[1]
    mesh = plsc.VectorSubcoreMesh(core_axis_name="c", subcore_axis_name="s")

    @functools.partial(
        pl.kernel,
        out_type=jax.ShapeDtypeStruct((total, d), jnp.float32),
        mesh=mesh,
        scratch_types=[
            pltpu.VMEM((chunks, _C), jnp.int32),
            pltpu.VMEM((_C, d), jnp.float32),
            pltpu.SemaphoreType.DMA,
        ],
        compiler_params=pltpu.CompilerParams(use_tc_tiling_on_sc=False),
    )
    def k(table_hbm, idx_hbm, out_hbm, idx_v, rows_v, sem):
        wid = lax.axis_index("s") * _NC + lax.axis_index("c")
        pltpu.sync_copy(idx_hbm.at[wid], idx_v)

        def body(j, carry):
            pltpu.async_copy(table_hbm.at[idx_v.at[j]], rows_v, sem).wait()
            base = (wid * chunks + j) * _C
            pltpu.sync_copy(rows_v, out_hbm.at[pl.ds(base, _C)])
            return carry

        lax.fori_loop(0, chunks, body, 0)

    return k(table2, idx3d)


def kernel(raw_seqs, embed_table, W, b):
    bsz, seq = raw_seqs.shape
    n, d = embed_table.shape
    total = bsz * seq
    table2 = _transform_table(embed_table, W, b)
    chunks = total // (_NW * _C)
    idx3d = raw_seqs.reshape(_NW, chunks, _C).astype(jnp.int32)
    out = _gather_rows(table2, idx3d, total, d)
    return out.reshape(bsz, seq, d)


# speedup vs baseline: 1.0657x; 1.0657x over previous
import functools

import jax
import jax.numpy as jnp
from jax import lax
from jax.experimental import pallas as pl
from jax.experimental.pallas import tpu as pltpu
from jax.experimental.pallas import tpu_sc as plsc

_NC = 2
_NS = 16
_NW = _NC * _NS
_C = 128


def _transform_table(table, W, b):
    n, d = table.shape
    blk = 1024

    def body(x_ref, w_ref, b_ref, o_ref):
        acc = jnp.dot(x_ref[...], w_ref[...], preferred_element_type=jnp.float32)
        o_ref[...] = jnp.maximum(acc + b_ref[...], 0.0)

    return pl.pallas_call(
        body,
        grid=(pl.cdiv(n, blk),),
        in_specs=[
            pl.BlockSpec((blk, d), lambda i: (i, 0)),
            pl.BlockSpec((d, d), lambda i: (0, 0)),
            pl.BlockSpec((1, d), lambda i: (0, 0)),
        ],
        out_specs=pl.BlockSpec((blk, d), lambda i: (i, 0)),
        out_shape=jax.ShapeDtypeStruct((n, d), jnp.float32),
    )(table, W, b.reshape(1, d))


_K = 4


def _gather_rows(table2, idx3d, total, d):
    chunks = idx3d.shape[1]
    groups = chunks // _K
    assert groups % 2 == 0 and groups * _K == chunks
    rows_g = _K * _C
    mesh = plsc.VectorSubcoreMesh(core_axis_name="c", subcore_axis_name="s")

    @functools.partial(
        pl.kernel,
        out_type=jax.ShapeDtypeStruct((total, d), jnp.float32),
        mesh=mesh,
        scratch_types=[
            pltpu.VMEM((chunks, _C), jnp.int32),
            pltpu.VMEM((rows_g, d), jnp.float32),
            pltpu.VMEM((rows_g, d), jnp.float32),
            pltpu.SemaphoreType.DMA,
            pltpu.SemaphoreType.DMA,
            pltpu.SemaphoreType.DMA,
            pltpu.SemaphoreType.DMA,
        ],
        compiler_params=pltpu.CompilerParams(use_tc_tiling_on_sc=False),
    )
    def k(table_hbm, idx_hbm, out_hbm, idx_v, buf0, buf1, sg0, sg1, sw0, sw1):
        wid = lax.axis_index("s") * _NC + lax.axis_index("c")
        w_base = wid * chunks * _C
        pltpu.sync_copy(idx_hbm.at[wid], idx_v)

        def fire_gathers(g, buf, sem):
            for i in range(_K):
                pltpu.async_copy(
                    table_hbm.at[idx_v.at[g * _K + i]],
                    buf.at[pl.ds(i * _C, _C)],
                    sem,
                )

        def drain_gathers(buf, sem):
            pltpu.make_async_copy(table_hbm.at[pl.ds(0, rows_g)], buf, sem).wait()

        def fire_writeback(g, buf, sem):
            pltpu.async_copy(
                buf, out_hbm.at[pl.ds(w_base + g * rows_g, rows_g)], sem
            )

        def drain_writeback(buf, sem):
            pltpu.make_async_copy(buf, out_hbm.at[pl.ds(0, rows_g)], sem).wait()

        fire_gathers(0, buf0, sg0)

        def body(t, carry):
            g0 = 2 * t
            g1 = 2 * t + 1
            drain_gathers(buf0, sg0)

            @pl.when(t > 0)
            def _():
                drain_writeback(buf1, sw1)

            fire_gathers(g1, buf1, sg1)
            fire_writeback(g0, buf0, sw0)

            drain_gathers(buf1, sg1)

            @pl.when(g1 + 1 < groups)
            def _():
                drain_writeback(buf0, sw0)
                fire_gathers(g1 + 1, buf0, sg0)

            fire_writeback(g1, buf1, sw1)
            return carry

        lax.fori_loop(0, groups // 2, body, 0)
        drain_writeback(buf0, sw0)
        drain_writeback(buf1, sw1)

    return k(table2, idx3d)


def kernel(raw_seqs, embed_table, W, b):
    bsz, seq = raw_seqs.shape
    n, d = embed_table.shape
    total = bsz * seq
    table2 = _transform_table(embed_table, W, b)
    chunks = total // (_NW * _C)
    idx3d = raw_seqs.reshape(_NW, chunks, _C).astype(jnp.int32)
    out = _gather_rows(table2, idx3d, total, d)
    return out.reshape(bsz, seq, d)
